# baseline (speedup 1.0000x reference)
import functools

import jax
import jax.numpy as jnp
from jax import lax
from jax.experimental import pallas as pl
from jax.experimental.pallas import tpu as pltpu
from jax.experimental.pallas import tpu_sc as plsc

_NC = 2
_NS = 16
_NW = _NC * _NS

_DIM = 32
_B_TOTAL = 16384 * 200
_IDX_COLS = 128
_IDX_ROWS = _B_TOTAL // _IDX_COLS
_ROWS_PER_W = _IDX_ROWS // _NW
_GROUPS = 8
_CHUNK = _GROUPS * _IDX_COLS
_N_CHUNKS = _ROWS_PER_W // _GROUPS
_N_PAIRS = _N_CHUNKS // 2

_mesh = plsc.VectorSubcoreMesh(
    core_axis_name="c", subcore_axis_name="s", num_cores=_NC, num_subcores=_NS
)


@functools.partial(
    pl.kernel,
    out_type=jax.ShapeDtypeStruct((_B_TOTAL, _DIM), jnp.float32),
    mesh=_mesh,
    scratch_types=[
        pltpu.VMEM((2, _GROUPS, _IDX_COLS), jnp.int32),
        pltpu.VMEM((2, _CHUNK, _DIM), jnp.float32),
        pltpu.SemaphoreType.DMA,
        pltpu.SemaphoreType.DMA,
        pltpu.SemaphoreType.DMA,
    ],
    compiler_params=pltpu.CompilerParams(use_tc_tiling_on_sc=False),
)
def _emb_lookup(pos_hbm, table_hbm, out_hbm, idx_v, rows_v, sem_i, sem_g, sem_o):
    wid = lax.axis_index("s") * _NC + lax.axis_index("c")
    base_row = wid * _ROWS_PER_W

    def idx_src(i):
        return pos_hbm.at[pl.ds(base_row + i * _GROUPS, _GROUPS), :]

    def out_dst(i):
        return out_hbm.at[pl.ds((base_row + i * _GROUPS) * _IDX_COLS, _CHUNK), :]

    def start_idx(i, s):
        pltpu.async_copy(idx_src(i), idx_v.at[s], sem_i)

    def wait_idx(s):
        pltpu.make_async_copy(idx_src(0), idx_v.at[s], sem_i).wait()

    def start_out(i, s):
        pltpu.async_copy(rows_v.at[s], out_dst(i), sem_o)

    def wait_out(s):
        pltpu.make_async_copy(rows_v.at[s], out_dst(0), sem_o).wait()

    start_idx(0, 0)
    start_idx(1, 1)

    @pl.loop(0, _N_PAIRS)
    def _pair(j):
        for s in (0, 1):
            i = 2 * j + s
            wait_idx(s)

            @pl.when(j > 0)
            def _():
                wait_out(s)

            gathers = [
                pltpu.async_copy(
                    table_hbm.at[idx_v.at[s, g]],
                    rows_v.at[s, pl.ds(g * _IDX_COLS, _IDX_COLS), :],
                    sem_g,
                )
                for g in range(_GROUPS)
            ]
            for c in gathers:
                c.wait()

            @pl.when(j < _N_PAIRS - 1)
            def _():
                start_idx(i + 2, s)

            start_out(i, s)

    wait_out(0)
    wait_out(1)


def kernel(positions, table):
    pos_flat = positions.reshape(_IDX_ROWS, _IDX_COLS)
    out = _emb_lookup(pos_flat, table)
    return out.reshape(positions.shape[0], positions.shape[1], _DIM)


# speedup vs baseline: 1.5533x; 1.5533x over previous
import functools

import jax
import jax.numpy as jnp
from jax import lax
from jax.experimental import pallas as pl
from jax.experimental.pallas import tpu as pltpu
from jax.experimental.pallas import tpu_sc as plsc

_NC = 2
_NS = 16
_NW = _NC * _NS

_DIM = 32
_B_TOTAL = 16384 * 200
_IDX_COLS = 128
_IDX_ROWS = _B_TOTAL // _IDX_COLS
_ROWS_PER_W = _IDX_ROWS // _NW
_GROUPS = 8
_CHUNK = _GROUPS * _IDX_COLS
_N_CHUNKS = _ROWS_PER_W // _GROUPS
_N_PAIRS = _N_CHUNKS // 2

_mesh = plsc.VectorSubcoreMesh(
    core_axis_name="c", subcore_axis_name="s", num_cores=_NC, num_subcores=_NS
)


@functools.partial(
    pl.kernel,
    out_type=jax.ShapeDtypeStruct((_B_TOTAL, _DIM), jnp.float32),
    mesh=_mesh,
    scratch_types=[
        pltpu.VMEM_SHARED((200, _DIM), jnp.float32),
        pltpu.VMEM((2, _GROUPS, _IDX_COLS), jnp.int32),
        pltpu.VMEM((2, _CHUNK, _DIM), jnp.float32),
        pltpu.SemaphoreType.DMA,
        pltpu.SemaphoreType.DMA,
        pltpu.SemaphoreType.DMA,
    ],
    compiler_params=pltpu.CompilerParams(use_tc_tiling_on_sc=False),
)
def _emb_lookup(pos_hbm, table_hbm, out_hbm, tab_sh, idx_v, rows_v, sem_i, sem_g, sem_o):
    sid = lax.axis_index("s")
    wid = sid * _NC + lax.axis_index("c")
    base_row = wid * _ROWS_PER_W

    @pl.when(sid == 0)
    def _():
        pltpu.sync_copy(table_hbm, tab_sh)

    plsc.subcore_barrier()

    def idx_src(i):
        return pos_hbm.at[pl.ds(base_row + i * _GROUPS, _GROUPS), :]

    def out_dst(i):
        return out_hbm.at[pl.ds((base_row + i * _GROUPS) * _IDX_COLS, _CHUNK), :]

    def start_idx(i, s):
        pltpu.async_copy(idx_src(i), idx_v.at[s], sem_i)

    def wait_idx(s):
        pltpu.make_async_copy(idx_src(0), idx_v.at[s], sem_i).wait()

    def start_out(i, s):
        pltpu.async_copy(rows_v.at[s], out_dst(i), sem_o)

    def wait_out(s):
        pltpu.make_async_copy(rows_v.at[s], out_dst(0), sem_o).wait()

    start_idx(0, 0)
    start_idx(1, 1)

    @pl.loop(0, _N_PAIRS)
    def _pair(j):
        for s in (0, 1):
            i = 2 * j + s
            wait_idx(s)

            @pl.when(j > 0)
            def _():
                wait_out(s)

            gathers = [
                pltpu.async_copy(
                    tab_sh.at[idx_v.at[s, g]],
                    rows_v.at[s, pl.ds(g * _IDX_COLS, _IDX_COLS), :],
                    sem_g,
                )
                for g in range(_GROUPS)
            ]
            for c in gathers:
                c.wait()

            @pl.when(j < _N_PAIRS - 1)
            def _():
                start_idx(i + 2, s)

            start_out(i, s)

    wait_out(0)
    wait_out(1)


def kernel(positions, table):
    pos_flat = positions.reshape(_IDX_ROWS, _IDX_COLS)
    out = _emb_lookup(pos_flat, table)
    return out.reshape(positions.shape[0], positions.shape[1], _DIM)
